# baseline (speedup 1.0000x reference)
import functools

import jax
import jax.numpy as jnp
from jax import lax
from jax.experimental import pallas as pl
from jax.experimental.pallas import tpu as pltpu
from jax.experimental.pallas import tpu_sc as plsc

N_NODES = 20000
N_EDGES = 100000
NPAD = 20480
EPAD = 102400
DUMMY = N_NODES
NW = 32
EW = EPAD // NW
CW = EW // 128
NSL = NPAD // 16

@functools.cache
def _sc_mesh():
    return plsc.VectorSubcoreMesh(core_axis_name="c", subcore_axis_name="s")

_DOT = functools.partial(jnp.dot, precision=jax.lax.Precision.HIGHEST,
                         preferred_element_type=jnp.float32)


def _sc_gather(table, idx2d, d):
    @functools.partial(
        pl.kernel, mesh=_sc_mesh(),
        compiler_params=pltpu.CompilerParams(use_tc_tiling_on_sc=False),
        out_type=jax.ShapeDtypeStruct((EPAD, d), jnp.float32),
        scratch_types=[pltpu.VMEM((32, 128), jnp.int32),
                       pltpu.VMEM((EW, d), jnp.float32)],
    )
    def k(table_hbm, idx_hbm, out_hbm, idx_v, rows_v):
        cid = lax.axis_index("c")
        sid = lax.axis_index("s")
        wid = sid * 2 + cid
        pltpu.sync_copy(idx_hbm.at[wid], idx_v)

        @pl.loop(0, CW)
        def _(j):
            pltpu.sync_copy(table_hbm.at[idx_v.at[j]],
                            rows_v.at[pl.ds(j * 128, 128)])

        pltpu.sync_copy(rows_v, out_hbm.at[pl.ds(wid * EW, EW)])

    return k(table, idx2d)


def _sc_scatter_add(msg, idx2d, zeros, d):
    @functools.partial(
        pl.kernel, mesh=_sc_mesh(),
        compiler_params=pltpu.CompilerParams(use_tc_tiling_on_sc=False),
        out_type=jax.ShapeDtypeStruct((2, NPAD, d), jnp.float32),
        scratch_types=[pltpu.VMEM((32, 128), jnp.int32),
                       pltpu.VMEM((128, d), jnp.float32),
                       pltpu.VMEM_SHARED((NPAD, d), jnp.float32)],
    )
    def k(msg_hbm, idx_hbm, zeros_hbm, out_hbm, idx_v, msg_v, acc_sh):
        cid = lax.axis_index("c")
        sid = lax.axis_index("s")
        wid = sid * 2 + cid
        pltpu.sync_copy(zeros_hbm.at[pl.ds(sid * NSL, NSL)],
                        acc_sh.at[pl.ds(sid * NSL, NSL)])
        pltpu.sync_copy(idx_hbm.at[wid], idx_v)
        plsc.subcore_barrier()

        @pl.loop(0, CW)
        def _(j):
            pltpu.sync_copy(msg_hbm.at[pl.ds(wid * EW + j * 128, 128)],
                            msg_v)
            pltpu.sync_copy(msg_v, acc_sh.at[idx_v.at[j]], add=True)

        plsc.subcore_barrier()
        pltpu.sync_copy(acc_sh.at[pl.ds(sid * NSL, NSL)],
                        out_hbm.at[cid, pl.ds(sid * NSL, NSL)])

    return k(msg, idx2d, zeros)


_ET = 1024
_NT = 1024


def _edge_messages(ea_p, xj, W1, b1, W2, b2, in_ch, out_ch):
    def body(ea_ref, xj_ref, w1_ref, b1_ref, w2_ref, b2_ref, out_ref):
        h = jnp.maximum(_DOT(ea_ref[...], w1_ref[...]) + b1_ref[...], 0.0)
        w = _DOT(h, w2_ref[...]) + b2_ref[...]
        acc = xj_ref[:, 0:1] * w[:, 0:out_ch]
        for i in range(1, in_ch):
            acc = acc + xj_ref[:, i:i + 1] * w[:, i * out_ch:(i + 1) * out_ch]
        out_ref[...] = acc

    return pl.pallas_call(
        body,
        grid=(EPAD // _ET,),
        in_specs=[
            pl.BlockSpec((_ET, 16), lambda i: (i, 0)),
            pl.BlockSpec((_ET, in_ch), lambda i: (i, 0)),
            pl.BlockSpec((16, 32), lambda i: (0, 0)),
            pl.BlockSpec((1, 32), lambda i: (0, 0)),
            pl.BlockSpec((32, in_ch * out_ch), lambda i: (0, 0)),
            pl.BlockSpec((1, in_ch * out_ch), lambda i: (0, 0)),
        ],
        out_specs=pl.BlockSpec((_ET, out_ch), lambda i: (i, 0)),
        out_shape=jax.ShapeDtypeStruct((EPAD, out_ch), jnp.float32),
        compiler_params=pltpu.CompilerParams(
            dimension_semantics=("parallel",)),
    )(ea_p, xj, W1, b1.reshape(1, -1), W2, b2.reshape(1, -1))


def _node_update(partials, xin, root, bias, in_ch, out_ch):
    def body(p_ref, x_ref, root_ref, bias_ref, out_ref):
        agg = p_ref[0] + p_ref[1]
        out_ref[...] = jnp.maximum(
            agg + _DOT(x_ref[...], root_ref[...]) + bias_ref[...], 0.0)

    return pl.pallas_call(
        body,
        grid=(NPAD // _NT,),
        in_specs=[
            pl.BlockSpec((2, _NT, out_ch), lambda i: (0, i, 0)),
            pl.BlockSpec((_NT, in_ch), lambda i: (i, 0)),
            pl.BlockSpec((in_ch, out_ch), lambda i: (0, 0)),
            pl.BlockSpec((1, out_ch), lambda i: (0, 0)),
        ],
        out_specs=pl.BlockSpec((_NT, out_ch), lambda i: (i, 0)),
        out_shape=jax.ShapeDtypeStruct((NPAD, out_ch), jnp.float32),
        compiler_params=pltpu.CompilerParams(
            dimension_semantics=("parallel",)),
    )(partials, xin, root, bias.reshape(1, -1))


def _final(partials, h1, root, bias, batch3d, fc1_W, fc1_b, out_W, out_b):
    ngrid = NPAD // _NT

    def body(p_ref, h1_ref, root_ref, bias_ref, batch_ref,
             fc1w_ref, fc1b_ref, outw_ref, outb_ref, out_ref, acc_ref):
        i = pl.program_id(0)

        @pl.when(i == 0)
        def _():
            acc_ref[...] = jnp.zeros_like(acc_ref)

        agg = p_ref[0] + p_ref[1]
        out2 = jnp.maximum(
            agg + _DOT(h1_ref[...], root_ref[...]) + bias_ref[...], 0.0)
        b = batch_ref[0, 0, :]
        onehot = (lax.broadcasted_iota(jnp.int32, (64, _NT), 0)
                  == b[None, :]).astype(jnp.float32)
        acc_ref[...] += _DOT(onehot, out2)

        @pl.when(i == ngrid - 1)
        def _():
            hh = jnp.maximum(_DOT(acc_ref[...], fc1w_ref[...])
                             + fc1b_ref[...], 0.0)
            out_ref[...] = _DOT(hh, outw_ref[...]) + outb_ref[...]

    return pl.pallas_call(
        body,
        grid=(ngrid,),
        in_specs=[
            pl.BlockSpec((2, _NT, 16), lambda i: (0, i, 0)),
            pl.BlockSpec((_NT, 32), lambda i: (i, 0)),
            pl.BlockSpec((32, 16), lambda i: (0, 0)),
            pl.BlockSpec((1, 16), lambda i: (0, 0)),
            pl.BlockSpec((1, 1, _NT), lambda i: (i, 0, 0)),
            pl.BlockSpec((16, 32), lambda i: (0, 0)),
            pl.BlockSpec((1, 32), lambda i: (0, 0)),
            pl.BlockSpec((32, 1), lambda i: (0, 0)),
            pl.BlockSpec((1, 1), lambda i: (0, 0)),
        ],
        out_specs=pl.BlockSpec((64, 1), lambda i: (0, 0)),
        out_shape=jax.ShapeDtypeStruct((64, 1), jnp.float32),
        scratch_shapes=[pltpu.VMEM((64, 16), jnp.float32)],
        compiler_params=pltpu.CompilerParams(
            dimension_semantics=("arbitrary",)),
    )(partials, h1, root, bias.reshape(1, -1), batch3d,
      fc1_W, fc1_b.reshape(1, -1), out_W, out_b.reshape(1, -1))


def kernel(x, edge_index, edge_attr, batch,
           c1_W1, c1_b1, c1_W2, c1_b2, c1_root, c1_bias,
           c2_W1, c2_b1, c2_W2, c2_b2, c2_root, c2_bias,
           fc1_W, fc1_b, out_W, out_b):
    src = edge_index[0].astype(jnp.int32)
    dst = edge_index[1].astype(jnp.int32)
    src2d = jnp.pad(
        jnp.pad(src, (0, EPAD - N_EDGES)).reshape(NW, CW, 128),
        ((0, 0), (0, 32 - CW), (0, 0)))
    dst2d = jnp.pad(
        jnp.pad(dst, (0, EPAD - N_EDGES),
                constant_values=DUMMY).reshape(NW, CW, 128),
        ((0, 0), (0, 32 - CW), (0, 0)), constant_values=DUMMY)
    ea_p = jnp.pad(edge_attr, ((0, EPAD - N_EDGES), (0, 0)))
    x_p = jnp.pad(x, ((0, NPAD - N_NODES), (0, 0)))
    batch3d = jnp.pad(batch.astype(jnp.int32), (0, NPAD - N_NODES),
                      constant_values=64).reshape(NPAD // _NT, 1, _NT)
    zeros32 = jnp.zeros((NPAD, 32), jnp.float32)
    zeros16 = jnp.zeros((NPAD, 16), jnp.float32)

    xj1 = _sc_gather(x, src2d, 16)
    msg1 = _edge_messages(ea_p, xj1, c1_W1, c1_b1, c1_W2, c1_b2, 16, 32)
    part1 = _sc_scatter_add(msg1, dst2d, zeros32, 32)
    out1 = _node_update(part1, x_p, c1_root, c1_bias, 16, 32)

    xj2 = _sc_gather(out1, src2d, 32)
    msg2 = _edge_messages(ea_p, xj2, c2_W1, c2_b1, c2_W2, c2_b2, 32, 16)
    part2 = _sc_scatter_add(msg2, dst2d, zeros16, 16)

    return _final(part2, out1, c2_root, c2_bias, batch3d,
                  fc1_W, fc1_b, out_W, out_b)


# speedup vs baseline: 2.8467x; 2.8467x over previous
import functools

import jax
import jax.numpy as jnp
from jax import lax
from jax.experimental import pallas as pl
from jax.experimental.pallas import tpu as pltpu
from jax.experimental.pallas import tpu_sc as plsc

N_NODES = 20000
N_EDGES = 100000
NPAD = 20480
EPAD = 102400
DUMMY = N_NODES
NW = 32
EW = EPAD // NW
CW = EW // 128
NSL = NPAD // 16

@functools.cache
def _sc_mesh():
    return plsc.VectorSubcoreMesh(core_axis_name="c", subcore_axis_name="s")

_DOT = functools.partial(jnp.dot, precision=jax.lax.Precision.HIGHEST,
                         preferred_element_type=jnp.float32)
_FDOT = functools.partial(jnp.dot, precision=jax.lax.Precision.DEFAULT,
                          preferred_element_type=jnp.float32)


def _sc_gather(table, idx2d, d):
    @functools.partial(
        pl.kernel, mesh=_sc_mesh(),
        compiler_params=pltpu.CompilerParams(use_tc_tiling_on_sc=False),
        out_type=jax.ShapeDtypeStruct((EPAD, d), jnp.float32),
        scratch_types=[pltpu.VMEM((32, 128), jnp.int32),
                       pltpu.VMEM((EW, d), jnp.float32),
                       pltpu.SemaphoreType.DMA],
    )
    def k(table_hbm, idx_hbm, out_hbm, idx_v, rows_v, sem):
        cid = lax.axis_index("c")
        sid = lax.axis_index("s")
        wid = sid * 2 + cid
        pltpu.sync_copy(idx_hbm.at[wid], idx_v)

        @pl.loop(0, CW)
        def _(j):
            pltpu.async_copy(table_hbm.at[idx_v.at[j]],
                             rows_v.at[pl.ds(j * 128, 128)], sem)

        @pl.loop(0, CW)
        def _(j):
            pltpu.make_async_copy(table_hbm.at[idx_v.at[j]],
                                  rows_v.at[pl.ds(j * 128, 128)], sem).wait()

        pltpu.sync_copy(rows_v, out_hbm.at[pl.ds(wid * EW, EW)])

    return k(table, idx2d)


def _sc_scatter_add(msg, idx2d, zeros, d):
    @functools.partial(
        pl.kernel, mesh=_sc_mesh(),
        compiler_params=pltpu.CompilerParams(use_tc_tiling_on_sc=False),
        out_type=jax.ShapeDtypeStruct((2, NPAD, d), jnp.float32),
        scratch_types=[pltpu.VMEM((32, 128), jnp.int32),
                       pltpu.VMEM((256, d), jnp.float32),
                       pltpu.VMEM_SHARED((NPAD, d), jnp.float32),
                       pltpu.SemaphoreType.DMA],
    )
    def k(msg_hbm, idx_hbm, zeros_hbm, out_hbm, idx_v, msg_v, acc_sh, sem):
        cid = lax.axis_index("c")
        sid = lax.axis_index("s")
        wid = sid * 2 + cid
        pltpu.sync_copy(zeros_hbm.at[pl.ds(sid * NSL, NSL)],
                        acc_sh.at[pl.ds(sid * NSL, NSL)])
        pltpu.sync_copy(idx_hbm.at[wid], idx_v)
        plsc.subcore_barrier()

        pltpu.async_copy(msg_hbm.at[pl.ds(wid * EW, 128)],
                         msg_v.at[pl.ds(0, 128)], sem)

        @pl.loop(0, CW)
        def _(j):
            buf = (j % 2) * 128
            nxt = j + 1

            @pl.when(nxt < CW)
            def _():
                pltpu.async_copy(
                    msg_hbm.at[pl.ds(wid * EW + nxt * 128, 128)],
                    msg_v.at[pl.ds((nxt % 2) * 128, 128)], sem)

            pltpu.make_async_copy(
                msg_hbm.at[pl.ds(wid * EW + j * 128, 128)],
                msg_v.at[pl.ds(buf, 128)], sem).wait()
            pltpu.sync_copy(msg_v.at[pl.ds(buf, 128)],
                            acc_sh.at[idx_v.at[j]], add=True)

        plsc.subcore_barrier()
        pltpu.sync_copy(acc_sh.at[pl.ds(sid * NSL, NSL)],
                        out_hbm.at[cid, pl.ds(sid * NSL, NSL)])

    return k(msg, idx2d, zeros)


_ET = 1024
_NT = 1024


def _edge_messages(ea_p, xj, W1, b1, W2, b2, in_ch, out_ch):
    kio = in_ch * out_ch
    lanes = jnp.arange(kio)
    rt = (jnp.arange(in_ch)[:, None] == (lanes[None, :] // out_ch)
          ).astype(jnp.float32)
    s = ((jnp.arange(128)[:, None] % out_ch) == jnp.arange(out_ch)[None, :]
         ).astype(jnp.float32)

    def body(ea_ref, xj_ref, w1_ref, b1_ref, w2_ref, b2_ref, rt_ref,
             s_ref, out_ref):
        h = jnp.maximum(_FDOT(ea_ref[...], w1_ref[...]) + b1_ref[...], 0.0)
        w = _FDOT(h, w2_ref[...]) + b2_ref[...]
        xe = _FDOT(xj_ref[...], rt_ref[...])
        p = xe * w
        acc = (p[:, 0:128] + p[:, 128:256] + p[:, 256:384] + p[:, 384:512])
        out_ref[...] = _FDOT(acc, s_ref[...])

    return pl.pallas_call(
        body,
        grid=(EPAD // _ET,),
        in_specs=[
            pl.BlockSpec((_ET, 16), lambda i: (i, 0)),
            pl.BlockSpec((_ET, in_ch), lambda i: (i, 0)),
            pl.BlockSpec((16, 32), lambda i: (0, 0)),
            pl.BlockSpec((1, 32), lambda i: (0, 0)),
            pl.BlockSpec((32, kio), lambda i: (0, 0)),
            pl.BlockSpec((1, kio), lambda i: (0, 0)),
            pl.BlockSpec((in_ch, kio), lambda i: (0, 0)),
            pl.BlockSpec((128, out_ch), lambda i: (0, 0)),
        ],
        out_specs=pl.BlockSpec((_ET, out_ch), lambda i: (i, 0)),
        out_shape=jax.ShapeDtypeStruct((EPAD, out_ch), jnp.float32),
        compiler_params=pltpu.CompilerParams(
            dimension_semantics=("parallel",)),
    )(ea_p, xj, W1, b1.reshape(1, -1), W2, b2.reshape(1, -1), rt, s)


def _node_update(partials, xin, root, bias, in_ch, out_ch):
    def body(p_ref, x_ref, root_ref, bias_ref, out_ref):
        agg = p_ref[0] + p_ref[1]
        out_ref[...] = jnp.maximum(
            agg + _DOT(x_ref[...], root_ref[...]) + bias_ref[...], 0.0)

    return pl.pallas_call(
        body,
        grid=(NPAD // _NT,),
        in_specs=[
            pl.BlockSpec((2, _NT, out_ch), lambda i: (0, i, 0)),
            pl.BlockSpec((_NT, in_ch), lambda i: (i, 0)),
            pl.BlockSpec((in_ch, out_ch), lambda i: (0, 0)),
            pl.BlockSpec((1, out_ch), lambda i: (0, 0)),
        ],
        out_specs=pl.BlockSpec((_NT, out_ch), lambda i: (i, 0)),
        out_shape=jax.ShapeDtypeStruct((NPAD, out_ch), jnp.float32),
        compiler_params=pltpu.CompilerParams(
            dimension_semantics=("parallel",)),
    )(partials, xin, root, bias.reshape(1, -1))


def _final(partials, h1, root, bias, batch3d, fc1_W, fc1_b, out_W, out_b):
    ngrid = NPAD // _NT

    def body(p_ref, h1_ref, root_ref, bias_ref, batch_ref,
             fc1w_ref, fc1b_ref, outw_ref, outb_ref, out_ref, acc_ref):
        i = pl.program_id(0)

        @pl.when(i == 0)
        def _():
            acc_ref[...] = jnp.zeros_like(acc_ref)

        agg = p_ref[0] + p_ref[1]
        out2 = jnp.maximum(
            agg + _DOT(h1_ref[...], root_ref[...]) + bias_ref[...], 0.0)
        b = batch_ref[0, 0, :]
        onehot = (lax.broadcasted_iota(jnp.int32, (64, _NT), 0)
                  == b[None, :]).astype(jnp.float32)
        acc_ref[...] += _DOT(onehot, out2)

        @pl.when(i == ngrid - 1)
        def _():
            hh = jnp.maximum(_DOT(acc_ref[...], fc1w_ref[...])
                             + fc1b_ref[...], 0.0)
            out_ref[...] = _DOT(hh, outw_ref[...]) + outb_ref[...]

    return pl.pallas_call(
        body,
        grid=(ngrid,),
        in_specs=[
            pl.BlockSpec((2, _NT, 16), lambda i: (0, i, 0)),
            pl.BlockSpec((_NT, 32), lambda i: (i, 0)),
            pl.BlockSpec((32, 16), lambda i: (0, 0)),
            pl.BlockSpec((1, 16), lambda i: (0, 0)),
            pl.BlockSpec((1, 1, _NT), lambda i: (i, 0, 0)),
            pl.BlockSpec((16, 32), lambda i: (0, 0)),
            pl.BlockSpec((1, 32), lambda i: (0, 0)),
            pl.BlockSpec((32, 1), lambda i: (0, 0)),
            pl.BlockSpec((1, 1), lambda i: (0, 0)),
        ],
        out_specs=pl.BlockSpec((64, 1), lambda i: (0, 0)),
        out_shape=jax.ShapeDtypeStruct((64, 1), jnp.float32),
        scratch_shapes=[pltpu.VMEM((64, 16), jnp.float32)],
        compiler_params=pltpu.CompilerParams(
            dimension_semantics=("arbitrary",)),
    )(partials, h1, root, bias.reshape(1, -1), batch3d,
      fc1_W, fc1_b.reshape(1, -1), out_W, out_b.reshape(1, -1))


def kernel(x, edge_index, edge_attr, batch,
           c1_W1, c1_b1, c1_W2, c1_b2, c1_root, c1_bias,
           c2_W1, c2_b1, c2_W2, c2_b2, c2_root, c2_bias,
           fc1_W, fc1_b, out_W, out_b):
    src = edge_index[0].astype(jnp.int32)
    dst = edge_index[1].astype(jnp.int32)
    src2d = jnp.pad(
        jnp.pad(src, (0, EPAD - N_EDGES)).reshape(NW, CW, 128),
        ((0, 0), (0, 32 - CW), (0, 0)))
    dst2d = jnp.pad(
        jnp.pad(dst, (0, EPAD - N_EDGES),
                constant_values=DUMMY).reshape(NW, CW, 128),
        ((0, 0), (0, 32 - CW), (0, 0)), constant_values=DUMMY)
    ea_p = jnp.pad(edge_attr, ((0, EPAD - N_EDGES), (0, 0)))
    x_p = jnp.pad(x, ((0, NPAD - N_NODES), (0, 0)))
    batch3d = jnp.pad(batch.astype(jnp.int32), (0, NPAD - N_NODES),
                      constant_values=64).reshape(NPAD // _NT, 1, _NT)
    zeros32 = jnp.zeros((NPAD, 32), jnp.float32)
    zeros16 = jnp.zeros((NPAD, 16), jnp.float32)

    xj1 = _sc_gather(x, src2d, 16)
    msg1 = _edge_messages(ea_p, xj1, c1_W1, c1_b1, c1_W2, c1_b2, 16, 32)
    part1 = _sc_scatter_add(msg1, dst2d, zeros32, 32)
    out1 = _node_update(part1, x_p, c1_root, c1_bias, 16, 32)

    xj2 = _sc_gather(out1, src2d, 32)
    msg2 = _edge_messages(ea_p, xj2, c2_W1, c2_b1, c2_W2, c2_b2, 32, 16)
    part2 = _sc_scatter_add(msg2, dst2d, zeros16, 16)

    return _final(part2, out1, c2_root, c2_bias, batch3d,
                  fc1_W, fc1_b, out_W, out_b)


# speedup vs baseline: 2.9377x; 1.0320x over previous
import functools

import jax
import jax.numpy as jnp
from jax import lax
from jax.experimental import pallas as pl
from jax.experimental.pallas import tpu as pltpu
from jax.experimental.pallas import tpu_sc as plsc

N_NODES = 20000
N_EDGES = 100000
NPAD = 20480
EPAD = 102400
DUMMY = N_NODES
NW = 32
EW = EPAD // NW
CW = EW // 128
NSL = NPAD // 16

@functools.cache
def _sc_mesh():
    return plsc.VectorSubcoreMesh(core_axis_name="c", subcore_axis_name="s")

_DOT = functools.partial(jnp.dot, precision=jax.lax.Precision.HIGHEST,
                         preferred_element_type=jnp.float32)
_FDOT = functools.partial(jnp.dot, precision=jax.lax.Precision.DEFAULT,
                          preferred_element_type=jnp.float32)


def _sc_gather(table, idx2d, d):
    @functools.partial(
        pl.kernel, mesh=_sc_mesh(),
        compiler_params=pltpu.CompilerParams(use_tc_tiling_on_sc=False),
        out_type=jax.ShapeDtypeStruct((EPAD, d), jnp.float32),
        scratch_types=[pltpu.VMEM((32, 128), jnp.int32),
                       pltpu.VMEM((EW, d), jnp.float32),
                       pltpu.SemaphoreType.DMA],
    )
    def k(table_hbm, idx_hbm, out_hbm, idx_v, rows_v, sem):
        cid = lax.axis_index("c")
        sid = lax.axis_index("s")
        wid = sid * 2 + cid
        pltpu.sync_copy(idx_hbm.at[wid], idx_v)

        @pl.loop(0, CW)
        def _(j):
            pltpu.async_copy(table_hbm.at[idx_v.at[j]],
                             rows_v.at[pl.ds(j * 128, 128)], sem)

        @pl.loop(0, CW)
        def _(j):
            pltpu.make_async_copy(table_hbm.at[idx_v.at[j]],
                                  rows_v.at[pl.ds(j * 128, 128)], sem).wait()

        pltpu.sync_copy(rows_v, out_hbm.at[pl.ds(wid * EW, EW)])

    return k(table, idx2d)


def _sc_scatter_add(msg, idx2d, zeros, d):
    @functools.partial(
        pl.kernel, mesh=_sc_mesh(),
        compiler_params=pltpu.CompilerParams(use_tc_tiling_on_sc=False),
        out_type=jax.ShapeDtypeStruct((2, NPAD, d), jnp.float32),
        scratch_types=[pltpu.VMEM((32, 128), jnp.int32),
                       pltpu.VMEM((256, d), jnp.float32),
                       pltpu.VMEM_SHARED((NPAD, d), jnp.float32),
                       pltpu.SemaphoreType.DMA],
    )
    def k(msg_hbm, idx_hbm, zeros_hbm, out_hbm, idx_v, msg_v, acc_sh, sem):
        cid = lax.axis_index("c")
        sid = lax.axis_index("s")
        wid = sid * 2 + cid
        pltpu.sync_copy(zeros_hbm.at[pl.ds(sid * NSL, NSL)],
                        acc_sh.at[pl.ds(sid * NSL, NSL)])
        pltpu.sync_copy(idx_hbm.at[wid], idx_v)
        plsc.subcore_barrier()

        pltpu.async_copy(msg_hbm.at[pl.ds(wid * EW, 128)],
                         msg_v.at[pl.ds(0, 128)], sem)

        @pl.loop(0, CW)
        def _(j):
            buf = (j % 2) * 128
            nxt = j + 1

            @pl.when(nxt < CW)
            def _():
                pltpu.async_copy(
                    msg_hbm.at[pl.ds(wid * EW + nxt * 128, 128)],
                    msg_v.at[pl.ds((nxt % 2) * 128, 128)], sem)

            pltpu.make_async_copy(
                msg_hbm.at[pl.ds(wid * EW + j * 128, 128)],
                msg_v.at[pl.ds(buf, 128)], sem).wait()
            pltpu.sync_copy(msg_v.at[pl.ds(buf, 128)],
                            acc_sh.at[idx_v.at[j]], add=True)

        plsc.subcore_barrier()
        pltpu.sync_copy(acc_sh.at[pl.ds(sid * NSL, NSL)],
                        out_hbm.at[cid, pl.ds(sid * NSL, NSL)])

    return k(msg, idx2d, zeros)


_ET = 1024
_NT = 1024


def _edge_messages(ea_p, xj, W1, b1, W2, b2, in_ch, out_ch):
    kio = in_ch * out_ch
    lanes = jnp.arange(kio)
    rt = (jnp.arange(in_ch)[:, None] == (lanes[None, :] // out_ch)
          ).astype(jnp.float32)
    s = ((jnp.arange(128)[:, None] % out_ch) == jnp.arange(out_ch)[None, :]
         ).astype(jnp.float32)

    def body(ea_ref, xj_ref, w1_ref, b1_ref, w2_ref, b2_ref, rt_ref,
             s_ref, out_ref):
        h = jnp.maximum(_FDOT(ea_ref[...], w1_ref[...]) + b1_ref[...], 0.0)
        w = _FDOT(h, w2_ref[...]) + b2_ref[...]
        xe = _FDOT(xj_ref[...], rt_ref[...])
        p = xe * w
        acc = (p[:, 0:128] + p[:, 128:256] + p[:, 256:384] + p[:, 384:512])
        msg = _FDOT(acc, s_ref[...])
        row = (lax.broadcasted_iota(jnp.int32, (_ET, 1), 0)
               + pl.program_id(0) * _ET)
        out_ref[...] = jnp.where(row < N_EDGES, msg, 0.0)

    return pl.pallas_call(
        body,
        grid=(EPAD // _ET,),
        in_specs=[
            pl.BlockSpec((_ET, 16),
                         lambda i: (jnp.minimum(i, (N_EDGES - 1) // _ET), 0)),
            pl.BlockSpec((_ET, in_ch), lambda i: (i, 0)),
            pl.BlockSpec((16, 32), lambda i: (0, 0)),
            pl.BlockSpec((1, 32), lambda i: (0, 0)),
            pl.BlockSpec((32, kio), lambda i: (0, 0)),
            pl.BlockSpec((1, kio), lambda i: (0, 0)),
            pl.BlockSpec((in_ch, kio), lambda i: (0, 0)),
            pl.BlockSpec((128, out_ch), lambda i: (0, 0)),
        ],
        out_specs=pl.BlockSpec((_ET, out_ch), lambda i: (i, 0)),
        out_shape=jax.ShapeDtypeStruct((EPAD, out_ch), jnp.float32),
        compiler_params=pltpu.CompilerParams(
            dimension_semantics=("parallel",)),
    )(ea_p, xj, W1, b1.reshape(1, -1), W2, b2.reshape(1, -1), rt, s)


def _node_update(partials, xin, root, bias, in_ch, out_ch):
    def body(p_ref, x_ref, root_ref, bias_ref, out_ref):
        agg = p_ref[0] + p_ref[1]
        out_ref[...] = jnp.maximum(
            agg + _DOT(x_ref[...], root_ref[...]) + bias_ref[...], 0.0)

    return pl.pallas_call(
        body,
        grid=(NPAD // _NT,),
        in_specs=[
            pl.BlockSpec((2, _NT, out_ch), lambda i: (0, i, 0)),
            pl.BlockSpec((_NT, in_ch), lambda i: (i, 0)),
            pl.BlockSpec((in_ch, out_ch), lambda i: (0, 0)),
            pl.BlockSpec((1, out_ch), lambda i: (0, 0)),
        ],
        out_specs=pl.BlockSpec((_NT, out_ch), lambda i: (i, 0)),
        out_shape=jax.ShapeDtypeStruct((N_NODES, out_ch), jnp.float32),
        compiler_params=pltpu.CompilerParams(
            dimension_semantics=("parallel",)),
    )(partials, xin, root, bias.reshape(1, -1))


def _final(partials, h1, root, bias, batch3d, fc1_W, fc1_b, out_W, out_b):
    ngrid = NPAD // _NT

    def body(p_ref, h1_ref, root_ref, bias_ref, batch_ref,
             fc1w_ref, fc1b_ref, outw_ref, outb_ref, out_ref, acc_ref):
        i = pl.program_id(0)

        @pl.when(i == 0)
        def _():
            acc_ref[...] = jnp.zeros_like(acc_ref)

        agg = p_ref[0] + p_ref[1]
        out2 = jnp.maximum(
            agg + _DOT(h1_ref[...], root_ref[...]) + bias_ref[...], 0.0)
        row = lax.broadcasted_iota(jnp.int32, (_NT, 1), 0) + i * _NT
        out2 = jnp.where(row < N_NODES, out2, 0.0)
        b = batch_ref[0, 0, :]
        onehot = (lax.broadcasted_iota(jnp.int32, (64, _NT), 0)
                  == b[None, :]).astype(jnp.float32)
        acc_ref[...] += _DOT(onehot, out2)

        @pl.when(i == ngrid - 1)
        def _():
            hh = jnp.maximum(_DOT(acc_ref[...], fc1w_ref[...])
                             + fc1b_ref[...], 0.0)
            out_ref[...] = _DOT(hh, outw_ref[...]) + outb_ref[...]

    return pl.pallas_call(
        body,
        grid=(ngrid,),
        in_specs=[
            pl.BlockSpec((2, _NT, 16), lambda i: (0, i, 0)),
            pl.BlockSpec((_NT, 32), lambda i: (i, 0)),
            pl.BlockSpec((32, 16), lambda i: (0, 0)),
            pl.BlockSpec((1, 16), lambda i: (0, 0)),
            pl.BlockSpec((1, 1, _NT), lambda i: (i, 0, 0)),
            pl.BlockSpec((16, 32), lambda i: (0, 0)),
            pl.BlockSpec((1, 32), lambda i: (0, 0)),
            pl.BlockSpec((32, 1), lambda i: (0, 0)),
            pl.BlockSpec((1, 1), lambda i: (0, 0)),
        ],
        out_specs=pl.BlockSpec((64, 1), lambda i: (0, 0)),
        out_shape=jax.ShapeDtypeStruct((64, 1), jnp.float32),
        scratch_shapes=[pltpu.VMEM((64, 16), jnp.float32)],
        compiler_params=pltpu.CompilerParams(
            dimension_semantics=("arbitrary",)),
    )(partials, h1, root, bias.reshape(1, -1), batch3d,
      fc1_W, fc1_b.reshape(1, -1), out_W, out_b.reshape(1, -1))


def kernel(x, edge_index, edge_attr, batch,
           c1_W1, c1_b1, c1_W2, c1_b2, c1_root, c1_bias,
           c2_W1, c2_b1, c2_W2, c2_b2, c2_root, c2_bias,
           fc1_W, fc1_b, out_W, out_b):
    src = edge_index[0].astype(jnp.int32)
    dst = edge_index[1].astype(jnp.int32)
    src2d = jnp.pad(
        jnp.pad(src, (0, EPAD - N_EDGES)).reshape(NW, CW, 128),
        ((0, 0), (0, 32 - CW), (0, 0)))
    dst2d = jnp.pad(
        jnp.pad(dst, (0, EPAD - N_EDGES),
                constant_values=DUMMY).reshape(NW, CW, 128),
        ((0, 0), (0, 32 - CW), (0, 0)), constant_values=DUMMY)
    batch3d = jnp.pad(batch.astype(jnp.int32), (0, NPAD - N_NODES),
                      constant_values=64).reshape(NPAD // _NT, 1, _NT)
    zeros32 = jnp.zeros((NPAD, 32), jnp.float32)
    zeros16 = jnp.zeros((NPAD, 16), jnp.float32)

    xj1 = _sc_gather(x, src2d, 16)
    msg1 = _edge_messages(edge_attr, xj1, c1_W1, c1_b1, c1_W2, c1_b2, 16, 32)
    part1 = _sc_scatter_add(msg1, dst2d, zeros32, 32)
    out1 = _node_update(part1, x, c1_root, c1_bias, 16, 32)

    xj2 = _sc_gather(out1, src2d, 32)
    msg2 = _edge_messages(edge_attr, xj2, c2_W1, c2_b1, c2_W2, c2_b2, 32, 16)
    part2 = _sc_scatter_add(msg2, dst2d, zeros16, 16)

    return _final(part2, out1, c2_root, c2_bias, batch3d,
                  fc1_W, fc1_b, out_W, out_b)
